# baseline (speedup 1.0000x reference)
import functools

import jax
import jax.numpy as jnp
from jax import lax
from jax.experimental import pallas as pl
from jax.experimental.pallas import tpu as pltpu
from jax.experimental.pallas import tpu_sc as plsc

N = 10000
D = 128
NPAD = 10240
E = 160000
EPAD = 163840
CHUNKS = 40
SENT = 10016
SEG = NPAD // 16

_MESH = plsc.VectorSubcoreMesh(
    core_axis_name="c", subcore_axis_name="s", num_cores=2, num_subcores=16
)


@functools.partial(
    pl.kernel,
    out_type=jax.ShapeDtypeStruct((2 * NPAD,), jnp.float32),
    mesh=_MESH,
    compiler_params=pltpu.CompilerParams(needs_layout_passes=False),
    scratch_types=[
        pltpu.VMEM((CHUNKS, 128), jnp.int32),
        pltpu.VMEM((CHUNKS, 128), jnp.float32),
        pltpu.VMEM((SEG,), jnp.float32),
        pltpu.VMEM_SHARED((NPAD,), jnp.float32),
    ],
)
def _deg_kernel(col_hbm, ones_hbm, out_hbm, colv, valv, stage, acc):
    c = lax.axis_index("c")
    s = lax.axis_index("s")
    wid = s * 2 + c
    seg = pl.ds(s * SEG, SEG)
    zv = jnp.zeros((16,), jnp.float32)
    for i in range(SEG // 16):
        stage[pl.ds(i * 16, 16)] = zv
    pltpu.sync_copy(stage, acc.at[seg])
    pltpu.sync_copy(col_hbm.at[pl.ds(wid * CHUNKS, CHUNKS), :], colv)
    pltpu.sync_copy(ones_hbm, valv)
    plsc.subcore_barrier()
    for j in range(CHUNKS):
        pltpu.sync_copy(valv.at[j], acc.at[colv.at[j]], add=True)
    plsc.subcore_barrier()
    pltpu.sync_copy(acc.at[seg], stage)
    pltpu.sync_copy(stage, out_hbm.at[pl.ds(c * NPAD + s * SEG, SEG)])


@functools.partial(
    pl.kernel,
    out_type=jax.ShapeDtypeStruct((2 * 3 * NPAD,), jnp.float32),
    mesh=_MESH,
    compiler_params=pltpu.CompilerParams(needs_layout_passes=False),
    scratch_types=[
        pltpu.VMEM((CHUNKS, 128), jnp.int32),
        pltpu.VMEM((CHUNKS, 128), jnp.int32),
        pltpu.VMEM((NPAD,), jnp.float32),
        pltpu.VMEM((NPAD,), jnp.float32),
        pltpu.VMEM((NPAD,), jnp.float32),
        pltpu.VMEM((CHUNKS, 128), jnp.float32),
        pltpu.VMEM((CHUNKS, 128), jnp.float32),
        pltpu.VMEM((CHUNKS, 128), jnp.float32),
        pltpu.VMEM((SEG,), jnp.float32),
        pltpu.VMEM((SEG,), jnp.float32),
        pltpu.VMEM((SEG,), jnp.float32),
        pltpu.VMEM((SEG,), jnp.float32),
        pltpu.SemaphoreType.DMA,
        pltpu.SemaphoreType.DMA,
        pltpu.VMEM_SHARED((NPAD,), jnp.float32),
        pltpu.VMEM_SHARED((NPAD,), jnp.float32),
        pltpu.VMEM_SHARED((NPAD,), jnp.float32),
    ],
)
def _round_kernel(row_hbm, col_hbm, u_hbm, out_hbm,
                  rowv, colv, ux, uy, uz, sx, sy, sz, stage, ost0, ost1, ost2,
                  sem_in, sem_sc, accx, accy, accz):
    c = lax.axis_index("c")
    s = lax.axis_index("s")
    wid = s * 2 + c
    seg = pl.ds(s * SEG, SEG)
    ins = [
        pltpu.async_copy(row_hbm.at[pl.ds(wid * CHUNKS, CHUNKS), :], rowv, sem_in),
        pltpu.async_copy(col_hbm.at[pl.ds(wid * CHUNKS, CHUNKS), :], colv, sem_in),
        pltpu.async_copy(u_hbm.at[pl.ds(0, NPAD)], ux, sem_in),
        pltpu.async_copy(u_hbm.at[pl.ds(NPAD, NPAD)], uy, sem_in),
        pltpu.async_copy(u_hbm.at[pl.ds(2 * NPAD, NPAD)], uz, sem_in),
    ]
    zv = jnp.zeros((16,), jnp.float32)
    for i in range(SEG // 16):
        stage[pl.ds(i * 16, 16)] = zv
    pltpu.sync_copy(stage, accx.at[seg])
    pltpu.sync_copy(stage, accy.at[seg])
    pltpu.sync_copy(stage, accz.at[seg])
    for d in ins:
        d.wait()
    plsc.subcore_barrier()
    pending = []
    for j in range(CHUNKS):
        for k in range(8):
            sl = pl.ds(k * 16, 16)
            r = rowv[j, sl]
            sx[j, sl] = plsc.load_gather(ux, [r])
            sy[j, sl] = plsc.load_gather(uy, [r])
            sz[j, sl] = plsc.load_gather(uz, [r])
        idx = colv.at[j]
        pending.append([
            pltpu.async_copy(sx.at[j], accx.at[idx], sem_sc, add=True),
            pltpu.async_copy(sy.at[j], accy.at[idx], sem_sc, add=True),
            pltpu.async_copy(sz.at[j], accz.at[idx], sem_sc, add=True),
        ])
        if len(pending) > 4:
            for d in pending.pop(0):
                d.wait()
    for grp in pending:
        for d in grp:
            d.wait()
    plsc.subcore_barrier()
    base = c * (3 * NPAD) + s * SEG
    outs = [
        pltpu.async_copy(accx.at[seg], ost0, sem_in),
        pltpu.async_copy(accy.at[seg], ost1, sem_in),
        pltpu.async_copy(accz.at[seg], ost2, sem_in),
    ]
    for d in outs:
        d.wait()
    outs = [
        pltpu.async_copy(ost0, out_hbm.at[pl.ds(base, SEG)], sem_in),
        pltpu.async_copy(ost1, out_hbm.at[pl.ds(base + NPAD, SEG)], sem_in),
        pltpu.async_copy(ost2, out_hbm.at[pl.ds(base + 2 * NPAD, SEG)], sem_in),
    ]
    for d in outs:
        d.wait()


def _pt_dinv(x_pad, wcat, deg2):
    def body(wcat_ref, x_ref, deg_ref, pt_ref, dinv_ref, u1_ref):
        ptb = lax.dot_general(
            wcat_ref[...], x_ref[...], (((0,), (1,)), ((), ())),
            preferred_element_type=jnp.float32)
        pt_ref[...] = ptb
        d = deg_ref[0:1, :] + deg_ref[1:2, :]
        dv = jnp.where(d > 0, lax.rsqrt(d), 0.0)
        dinv_ref[...] = dv
        u1_ref[...] = ptb[9:12, :] * dv

    return pl.pallas_call(
        body,
        grid=(8,),
        in_specs=[
            pl.BlockSpec((128, 16), lambda i: (0, 0)),
            pl.BlockSpec((1280, 128), lambda i: (i, 0)),
            pl.BlockSpec((2, 1280), lambda i: (0, i)),
        ],
        out_specs=[
            pl.BlockSpec((16, 1280), lambda i: (0, i)),
            pl.BlockSpec((1, 1280), lambda i: (0, i)),
            pl.BlockSpec((3, 1280), lambda i: (0, i)),
        ],
        out_shape=[
            jax.ShapeDtypeStruct((16, NPAD), jnp.float32),
            jax.ShapeDtypeStruct((1, NPAD), jnp.float32),
            jax.ShapeDtypeStruct((3, NPAD), jnp.float32),
        ],
    )(wcat, x_pad, deg2)


def _finalize(parts, ptk, dinv1):
    def body(p_ref, pt_ref, dv_ref, o_ref):
        dv = dv_ref[...]
        o_ref[...] = dv * (dv * (p_ref[0] + p_ref[1]) + pt_ref[...])

    return pl.pallas_call(
        body,
        out_shape=jax.ShapeDtypeStruct((3, NPAD), jnp.float32),
    )(parts, ptk, dinv1)


def _finalize_z(parts, ptz, dinv1, bias):
    def body(p_ref, pt_ref, dv_ref, b_ref, o_ref):
        zcm = dv_ref[...] * (p_ref[0] + p_ref[1]) + pt_ref[...] + b_ref[...]
        o_ref[...] = jnp.swapaxes(zcm, 0, 1)

    return pl.pallas_call(
        body,
        out_shape=jax.ShapeDtypeStruct((NPAD, 3), jnp.float32),
    )(parts, ptz, dinv1, bias)


def _decode(zt, w1, b1, w2, b2, w3, b3):
    TR = 400

    def body(zr_ref, zc_ref, w1_ref, b1_ref, w2_ref, b2_ref, w3_ref, b3_ref,
             a_ref, xr_ref):
        zr = zr_ref[...]
        acc = lax.dot_general(zr, zc_ref[...], (((1,), (1,)), ((), ())),
                              preferred_element_type=jnp.float32)
        a_ref[...] = 1.0 / (1.0 + jnp.exp(-acc))
        h = jnp.dot(zr, w1_ref[...],
                    preferred_element_type=jnp.float32) + b1_ref[...]
        h = jnp.where(h >= 0, h, 0.01 * h)
        h = jnp.dot(h, w2_ref[...],
                    preferred_element_type=jnp.float32) + b2_ref[...]
        h = jnp.where(h >= 0, h, 0.01 * h)
        xr = jnp.dot(h, w3_ref[...],
                     preferred_element_type=jnp.float32) + b3_ref[...]
        xr_ref[...] = 1.0 / (1.0 + jnp.exp(-xr))

    return pl.pallas_call(
        body,
        grid=(N // TR,),
        in_specs=[
            pl.BlockSpec((TR, 3), lambda i: (i, 0)),
            pl.BlockSpec((N, 3), lambda i: (0, 0)),
            pl.BlockSpec((3, 128), lambda i: (0, 0)),
            pl.BlockSpec((1, 128), lambda i: (0, 0)),
            pl.BlockSpec((128, 256), lambda i: (0, 0)),
            pl.BlockSpec((1, 256), lambda i: (0, 0)),
            pl.BlockSpec((256, 128), lambda i: (0, 0)),
            pl.BlockSpec((1, 128), lambda i: (0, 0)),
        ],
        out_specs=[
            pl.BlockSpec((TR, N), lambda i: (i, 0)),
            pl.BlockSpec((TR, 128), lambda i: (i, 0)),
        ],
        out_shape=[
            jax.ShapeDtypeStruct((N, N), jnp.float32),
            jax.ShapeDtypeStruct((N, D), jnp.float32),
        ],
    )(zt, zt, w1, b1, w2, b2, w3, b3)


def kernel(x, e, enc_w0, enc_w1, enc_w2, enc_w3, enc_b,
           dec_w1, dec_b1, dec_w2, dec_b2, dec_w3, dec_b3):
    x_pad = jnp.pad(x, ((0, NPAD - N), (0, 0)))
    wcat = jnp.pad(
        jnp.concatenate([enc_w0, enc_w1, enc_w2, enc_w3], axis=1),
        ((0, 0), (0, 4)))
    row2d = jnp.pad(e[0], (0, EPAD - E)).reshape(EPAD // 128, 128)
    col2d = jnp.pad(e[1], (0, EPAD - E),
                    constant_values=SENT).reshape(EPAD // 128, 128)
    ones_c = jnp.ones((CHUNKS, 128), jnp.float32)

    deg_parts = _deg_kernel(col2d, ones_c)
    pt, dinv1, u1 = _pt_dinv(x_pad, wcat, deg_parts.reshape(2, NPAD))

    u = u1.reshape(3 * NPAD)
    for k in (2, 1):
        parts = _round_kernel(row2d, col2d, u)
        u = _finalize(parts.reshape(2, 3, NPAD),
                      pt[3 * k:3 * k + 3, :], dinv1).reshape(3 * NPAD)
    parts = _round_kernel(row2d, col2d, u)
    zt = _finalize_z(parts.reshape(2, 3, NPAD), pt[0:3, :], dinv1,
                     enc_b.reshape(3, 1))

    a_out, xr = _decode(zt[:N, :], dec_w1, dec_b1.reshape(1, 128),
                        dec_w2, dec_b2.reshape(1, 256),
                        dec_w3, dec_b3.reshape(1, 128))
    return xr, a_out


# speedup vs baseline: 1.0255x; 1.0255x over previous
import functools

import jax
import jax.numpy as jnp
from jax import lax
from jax.experimental import pallas as pl
from jax.experimental.pallas import tpu as pltpu
from jax.experimental.pallas import tpu_sc as plsc

N = 10000
D = 128
NPAD = 10240
E = 160000
EPAD = 163840
CHUNKS = 40
SENT = 10016
SEG = NPAD // 16

_MESH = plsc.VectorSubcoreMesh(
    core_axis_name="c", subcore_axis_name="s", num_cores=2, num_subcores=16
)


@functools.partial(
    pl.kernel,
    out_type=jax.ShapeDtypeStruct((2 * NPAD,), jnp.float32),
    mesh=_MESH,
    compiler_params=pltpu.CompilerParams(needs_layout_passes=False),
    scratch_types=[
        pltpu.VMEM((CHUNKS, 128), jnp.int32),
        pltpu.VMEM((CHUNKS, 128), jnp.float32),
        pltpu.VMEM((SEG,), jnp.float32),
        pltpu.VMEM_SHARED((NPAD,), jnp.float32),
    ],
)
def _deg_kernel(col_hbm, ones_hbm, out_hbm, colv, valv, stage, acc):
    c = lax.axis_index("c")
    s = lax.axis_index("s")
    wid = s * 2 + c
    seg = pl.ds(s * SEG, SEG)
    zv = jnp.zeros((16,), jnp.float32)
    for i in range(SEG // 16):
        stage[pl.ds(i * 16, 16)] = zv
    pltpu.sync_copy(stage, acc.at[seg])
    pltpu.sync_copy(col_hbm.at[pl.ds(wid * CHUNKS, CHUNKS), :], colv)
    pltpu.sync_copy(ones_hbm, valv)
    plsc.subcore_barrier()
    for j in range(CHUNKS):
        pltpu.sync_copy(valv.at[j], acc.at[colv.at[j]], add=True)
    plsc.subcore_barrier()
    pltpu.sync_copy(acc.at[seg], stage)
    pltpu.sync_copy(stage, out_hbm.at[pl.ds(c * NPAD + s * SEG, SEG)])


@functools.partial(
    pl.kernel,
    out_type=jax.ShapeDtypeStruct((2 * 3 * NPAD,), jnp.float32),
    mesh=_MESH,
    compiler_params=pltpu.CompilerParams(needs_layout_passes=False),
    scratch_types=[
        pltpu.VMEM((CHUNKS, 128), jnp.int32),
        pltpu.VMEM((CHUNKS, 128), jnp.int32),
        pltpu.VMEM((NPAD,), jnp.float32),
        pltpu.VMEM((NPAD,), jnp.float32),
        pltpu.VMEM((NPAD,), jnp.float32),
        pltpu.VMEM((CHUNKS, 128), jnp.float32),
        pltpu.VMEM((CHUNKS, 128), jnp.float32),
        pltpu.VMEM((CHUNKS, 128), jnp.float32),
        pltpu.VMEM((SEG,), jnp.float32),
        pltpu.VMEM((SEG,), jnp.float32),
        pltpu.VMEM((SEG,), jnp.float32),
        pltpu.VMEM((SEG,), jnp.float32),
        pltpu.SemaphoreType.DMA,
        pltpu.SemaphoreType.DMA,
        pltpu.VMEM_SHARED((NPAD,), jnp.float32),
        pltpu.VMEM_SHARED((NPAD,), jnp.float32),
        pltpu.VMEM_SHARED((NPAD,), jnp.float32),
    ],
)
def _round_kernel(row_hbm, col_hbm, u_hbm, out_hbm,
                  rowv, colv, ux, uy, uz, sx, sy, sz, stage, ost0, ost1, ost2,
                  sem_in, sem_sc, accx, accy, accz):
    c = lax.axis_index("c")
    s = lax.axis_index("s")
    wid = s * 2 + c
    seg = pl.ds(s * SEG, SEG)
    ins = [
        pltpu.async_copy(row_hbm.at[pl.ds(wid * CHUNKS, CHUNKS), :], rowv, sem_in),
        pltpu.async_copy(col_hbm.at[pl.ds(wid * CHUNKS, CHUNKS), :], colv, sem_in),
        pltpu.async_copy(u_hbm.at[pl.ds(0, NPAD)], ux, sem_in),
        pltpu.async_copy(u_hbm.at[pl.ds(NPAD, NPAD)], uy, sem_in),
        pltpu.async_copy(u_hbm.at[pl.ds(2 * NPAD, NPAD)], uz, sem_in),
    ]
    zv = jnp.zeros((16,), jnp.float32)
    for i in range(SEG // 16):
        stage[pl.ds(i * 16, 16)] = zv
    pltpu.sync_copy(stage, accx.at[seg])
    pltpu.sync_copy(stage, accy.at[seg])
    pltpu.sync_copy(stage, accz.at[seg])
    for d in ins:
        d.wait()
    plsc.subcore_barrier()
    pending = []
    for j in range(CHUNKS):
        for k in range(8):
            sl = pl.ds(k * 16, 16)
            r = rowv[j, sl]
            sx[j, sl] = plsc.load_gather(ux, [r])
            sy[j, sl] = plsc.load_gather(uy, [r])
            sz[j, sl] = plsc.load_gather(uz, [r])
        idx = colv.at[j]
        pending.append([
            pltpu.async_copy(sx.at[j], accx.at[idx], sem_sc, add=True),
            pltpu.async_copy(sy.at[j], accy.at[idx], sem_sc, add=True),
            pltpu.async_copy(sz.at[j], accz.at[idx], sem_sc, add=True),
        ])
        if len(pending) > 4:
            for d in pending.pop(0):
                d.wait()
    for grp in pending:
        for d in grp:
            d.wait()
    plsc.subcore_barrier()
    base = c * (3 * NPAD) + s * SEG
    outs = [
        pltpu.async_copy(accx.at[seg], ost0, sem_in),
        pltpu.async_copy(accy.at[seg], ost1, sem_in),
        pltpu.async_copy(accz.at[seg], ost2, sem_in),
    ]
    for d in outs:
        d.wait()
    outs = [
        pltpu.async_copy(ost0, out_hbm.at[pl.ds(base, SEG)], sem_in),
        pltpu.async_copy(ost1, out_hbm.at[pl.ds(base + NPAD, SEG)], sem_in),
        pltpu.async_copy(ost2, out_hbm.at[pl.ds(base + 2 * NPAD, SEG)], sem_in),
    ]
    for d in outs:
        d.wait()


def _pt(x_pad, wcat):
    def body(wcat_ref, x_ref, pt_ref):
        pt_ref[...] = lax.dot_general(
            wcat_ref[...], x_ref[...], (((0,), (1,)), ((), ())),
            preferred_element_type=jnp.float32)

    return pl.pallas_call(
        body,
        grid=(8,),
        in_specs=[
            pl.BlockSpec((128, 16), lambda i: (0, 0)),
            pl.BlockSpec((1280, 128), lambda i: (i, 0)),
        ],
        out_specs=pl.BlockSpec((16, 1280), lambda i: (0, i)),
        out_shape=jax.ShapeDtypeStruct((16, NPAD), jnp.float32),
    )(wcat, x_pad)


def _dinv_u1(deg2, pt):
    def body(deg_ref, pt_ref, dinv_ref, u1_ref):
        d = deg_ref[0:1, :] + deg_ref[1:2, :]
        dv = jnp.where(d > 0, lax.rsqrt(d), 0.0)
        dinv_ref[...] = dv
        u1_ref[...] = pt_ref[9:12, :] * dv

    return pl.pallas_call(
        body,
        out_shape=[
            jax.ShapeDtypeStruct((1, NPAD), jnp.float32),
            jax.ShapeDtypeStruct((3, NPAD), jnp.float32),
        ],
    )(deg2, pt)


def _finalize(parts, ptk, dinv1):
    def body(p_ref, pt_ref, dv_ref, o_ref):
        dv = dv_ref[...]
        o_ref[...] = dv * (dv * (p_ref[0] + p_ref[1]) + pt_ref[...])

    return pl.pallas_call(
        body,
        out_shape=jax.ShapeDtypeStruct((3, NPAD), jnp.float32),
    )(parts, ptk, dinv1)


def _finalize_z(parts, ptz, dinv1, bias):
    def body(p_ref, pt_ref, dv_ref, b_ref, o_ref):
        zcm = dv_ref[...] * (p_ref[0] + p_ref[1]) + pt_ref[...] + b_ref[...]
        o_ref[...] = jnp.swapaxes(zcm, 0, 1)

    return pl.pallas_call(
        body,
        out_shape=jax.ShapeDtypeStruct((NPAD, 3), jnp.float32),
    )(parts, ptz, dinv1, bias)


def _decode(zt, w1, b1, w2, b2, w3, b3):
    TR = 400

    def body(zr_ref, zc_ref, w1_ref, b1_ref, w2_ref, b2_ref, w3_ref, b3_ref,
             a_ref, xr_ref):
        zr = zr_ref[...]
        acc = lax.dot_general(zr, zc_ref[...], (((1,), (1,)), ((), ())),
                              preferred_element_type=jnp.float32)
        a_ref[...] = 1.0 / (1.0 + jnp.exp(-acc))
        h = jnp.dot(zr, w1_ref[...],
                    preferred_element_type=jnp.float32) + b1_ref[...]
        h = jnp.where(h >= 0, h, 0.01 * h)
        h = jnp.dot(h, w2_ref[...],
                    preferred_element_type=jnp.float32) + b2_ref[...]
        h = jnp.where(h >= 0, h, 0.01 * h)
        xr = jnp.dot(h, w3_ref[...],
                     preferred_element_type=jnp.float32) + b3_ref[...]
        xr_ref[...] = 1.0 / (1.0 + jnp.exp(-xr))

    return pl.pallas_call(
        body,
        grid=(N // TR,),
        in_specs=[
            pl.BlockSpec((TR, 3), lambda i: (i, 0)),
            pl.BlockSpec((N, 3), lambda i: (0, 0)),
            pl.BlockSpec((3, 128), lambda i: (0, 0)),
            pl.BlockSpec((1, 128), lambda i: (0, 0)),
            pl.BlockSpec((128, 256), lambda i: (0, 0)),
            pl.BlockSpec((1, 256), lambda i: (0, 0)),
            pl.BlockSpec((256, 128), lambda i: (0, 0)),
            pl.BlockSpec((1, 128), lambda i: (0, 0)),
        ],
        out_specs=[
            pl.BlockSpec((TR, N), lambda i: (i, 0)),
            pl.BlockSpec((TR, 128), lambda i: (i, 0)),
        ],
        out_shape=[
            jax.ShapeDtypeStruct((N, N), jnp.float32),
            jax.ShapeDtypeStruct((N, D), jnp.float32),
        ],
    )(zt, zt, w1, b1, w2, b2, w3, b3)


def kernel(x, e, enc_w0, enc_w1, enc_w2, enc_w3, enc_b,
           dec_w1, dec_b1, dec_w2, dec_b2, dec_w3, dec_b3):
    x_pad = jnp.pad(x, ((0, NPAD - N), (0, 0)))
    wcat = jnp.pad(
        jnp.concatenate([enc_w0, enc_w1, enc_w2, enc_w3], axis=1),
        ((0, 0), (0, 4)))
    row2d = jnp.pad(e[0], (0, EPAD - E)).reshape(EPAD // 128, 128)
    col2d = jnp.pad(e[1], (0, EPAD - E),
                    constant_values=SENT).reshape(EPAD // 128, 128)
    ones_c = jnp.ones((CHUNKS, 128), jnp.float32)

    pt = _pt(x_pad, wcat)
    deg_parts = _deg_kernel(col2d, ones_c)
    dinv1, u1 = _dinv_u1(deg_parts.reshape(2, NPAD), pt)

    u = u1.reshape(3 * NPAD)
    for k in (2, 1):
        parts = _round_kernel(row2d, col2d, u)
        u = _finalize(parts.reshape(2, 3, NPAD),
                      pt[3 * k:3 * k + 3, :], dinv1).reshape(3 * NPAD)
    parts = _round_kernel(row2d, col2d, u)
    zt = _finalize_z(parts.reshape(2, 3, NPAD), pt[0:3, :], dinv1,
                     enc_b.reshape(3, 1))

    a_out, xr = _decode(zt[:N, :], dec_w1, dec_b1.reshape(1, 128),
                        dec_w2, dec_b2.reshape(1, 256),
                        dec_w3, dec_b3.reshape(1, 128))
    return xr, a_out
